# Initial kernel scaffold; baseline (speedup 1.0000x reference)
import jax
import jax.numpy as jnp
from jax.experimental import pallas as pl


def kernel(x, edge_index, edge_attr, ppi, labels, eps0, W1_0, b1_0, W2_0, b2_0, eps1, W1_1, b1_1, W2_1, b2_1, W_out, b_out):
    raise NotImplementedError("write your pallas kernel here")


# speedup vs baseline: 1.7690x; 1.7690x over previous
import functools

import jax
import jax.numpy as jnp
from jax import lax
from jax.experimental import pallas as pl
from jax.experimental.pallas import tpu as pltpu
from jax.experimental.pallas import tpu_sc as plsc

_N = 10000
_E = 160000
_D = 256
_B = 4096
_C = 7

_NC = 2
_NS = 16
_L = 16

_DH = _D // 2
_SPR = 10240
_G = 80
_EPW = _E // _NS
_CHUNKS = _EPW // _G

_mesh = plsc.VectorSubcoreMesh(
    core_axis_name="c", subcore_axis_name="s",
    num_cores=_NC, num_subcores=_NS)


@functools.partial(
    pl.kernel,
    out_type=[jax.ShapeDtypeStruct((_N, _DH), jnp.float32),
              jax.ShapeDtypeStruct((_N, _DH), jnp.float32)],
    mesh=_mesh,
    scratch_types=[
        pltpu.VMEM((2, _G), jnp.int32),
        pltpu.VMEM((2, _G), jnp.int32),
        pltpu.VMEM((2, _G, _DH), jnp.float32),
        pltpu.VMEM((2, _G, _DH), jnp.float32),
        pltpu.VMEM_SHARED((_SPR, _DH), jnp.float32),
        pltpu.SemaphoreType.DMA((2,)),
        pltpu.SemaphoreType.DMA((2,)),
    ],
)
def _agg(hl_hbm, hr_hbm, src_hbm, dst_hbm, ea_hbm, outl_hbm, outr_hbm,
         srcv, dstv, hbuf, mbuf, accs, gsem, esem):
    c = lax.axis_index("c")
    s = lax.axis_index("s")

    def _zrow(r, carry):
        for j in range(_DH // _L):
            mbuf[0, r, pl.ds(j * _L, _L)] = jnp.zeros((_L,), jnp.float32)
        return carry
    lax.fori_loop(0, _G, _zrow, 0)
    for k in range(_SPR // _NS // _G):
        pltpu.sync_copy(mbuf.at[0],
                        accs.at[pl.ds(s * (_SPR // _NS) + k * _G, _G)])
    plsc.subcore_barrier()

    def _issue(t, p):
        base = s * _EPW + t * _G
        pltpu.sync_copy(src_hbm.at[pl.ds(base, _G)], srcv.at[p])
        pltpu.sync_copy(dst_hbm.at[pl.ds(base, _G)], dstv.at[p])

        @pl.when(c == 0)
        def _():
            pltpu.async_copy(hl_hbm.at[srcv.at[p]], hbuf.at[p], gsem.at[p])
            pltpu.async_copy(ea_hbm.at[pl.ds(base, _G), pl.ds(0, _DH)],
                             mbuf.at[p], esem.at[p])

        @pl.when(c == 1)
        def _():
            pltpu.async_copy(hr_hbm.at[srcv.at[p]], hbuf.at[p], gsem.at[p])
            pltpu.async_copy(ea_hbm.at[pl.ds(base, _G), pl.ds(_DH, _DH)],
                             mbuf.at[p], esem.at[p])

    def _waitload(p):
        pltpu.make_async_copy(hl_hbm.at[srcv.at[p]], hbuf.at[p],
                              gsem.at[p]).wait()
        pltpu.make_async_copy(ea_hbm.at[pl.ds(0, _G), pl.ds(0, _DH)],
                              mbuf.at[p], esem.at[p]).wait()

    _issue(0, 0)

    def _body(t, carry):
        p = lax.rem(t, 2)
        q = 1 - p

        @pl.when(t + 1 < _CHUNKS)
        def _():
            _issue(t + 1, q)

        _waitload(p)

        def _mrow(r, cc):
            for j in range(_DH // _L):
                sl = pl.ds(j * _L, _L)
                mbuf[p, r, sl] = jnp.maximum(hbuf[p, r, sl] + mbuf[p, r, sl],
                                             0.0)
            return cc
        lax.fori_loop(0, _G, _mrow, 0)

        pltpu.sync_copy(mbuf.at[p], accs.at[dstv.at[p]], add=True)
        return carry
    lax.fori_loop(0, _CHUNKS, _body, 0)
    plsc.subcore_barrier()

    def _copy_out(out_hbm):
        @pl.when(s < 15)
        def _():
            off = s * 624
            pltpu.sync_copy(accs.at[pl.ds(off, 624)],
                            out_hbm.at[pl.ds(off, 624)])

        @pl.when(s == 15)
        def _():
            pltpu.sync_copy(accs.at[pl.ds(15 * 624, 640)],
                            out_hbm.at[pl.ds(15 * 624, 640)])

    @pl.when(c == 0)
    def _():
        _copy_out(outl_hbm)

    @pl.when(c == 1)
    def _():
        _copy_out(outr_hbm)


_RPW = (2 * _B) // (_NC * _NS)


@functools.partial(
    pl.kernel,
    out_type=[jax.ShapeDtypeStruct((2 * _B, _DH), jnp.float32),
              jax.ShapeDtypeStruct((2 * _B, _DH), jnp.float32)],
    mesh=_mesh,
    scratch_types=[
        pltpu.VMEM((_RPW,), jnp.int32),
        pltpu.VMEM((_RPW, _DH), jnp.float32),
        pltpu.VMEM((_RPW, _DH), jnp.float32),
        pltpu.SemaphoreType.DMA,
        pltpu.SemaphoreType.DMA,
    ],
)
def _pair_gather(hl_hbm, hr_hbm, idx_hbm, outl_hbm, outr_hbm,
                 idxv, rowsl, rowsr, seml, semr):
    c = lax.axis_index("c")
    s = lax.axis_index("s")
    base = (s * _NC + c) * _RPW
    pltpu.sync_copy(idx_hbm.at[pl.ds(base, _RPW)], idxv)
    cl = pltpu.async_copy(hl_hbm.at[idxv], rowsl, seml)
    cr = pltpu.async_copy(hr_hbm.at[idxv], rowsr, semr)
    cl.wait()
    cr.wait()
    pltpu.sync_copy(rowsl, outl_hbm.at[pl.ds(base, _RPW)])
    pltpu.sync_copy(rowsr, outr_hbm.at[pl.ds(base, _RPW)])


_R = 400


def _mlp_body(relu_out, eps_ref, hl_ref, hr_ref, al_ref, ar_ref,
              w1_ref, b1_ref, w2_ref, b2_ref, outl_ref, outr_ref):
    h = jnp.concatenate([hl_ref[...], hr_ref[...]], axis=1)
    agg = jnp.concatenate([al_ref[...], ar_ref[...]], axis=1)
    z = (1.0 + eps_ref[0, 0]) * h + agg
    z = jnp.maximum(
        jnp.dot(z, w1_ref[...], preferred_element_type=jnp.float32)
        + b1_ref[...], 0.0)
    z = jnp.dot(z, w2_ref[...], preferred_element_type=jnp.float32) + b2_ref[...]
    if relu_out:
        z = jnp.maximum(z, 0.0)
    outl_ref[...] = z[:, :_DH]
    outr_ref[...] = z[:, _DH:]


def _mlp(hl, hr, al, ar, eps, w1, b1, w2, b2, relu_out):
    half = pl.BlockSpec((_R, _DH), lambda i: (i, 0))
    return pl.pallas_call(
        functools.partial(_mlp_body, relu_out),
        grid=(_N // _R,),
        in_specs=[
            pl.BlockSpec(memory_space=pltpu.SMEM),
            half, half, half, half,
            pl.BlockSpec((_D, _D), lambda i: (0, 0)),
            pl.BlockSpec((1, _D), lambda i: (0, 0)),
            pl.BlockSpec((_D, _D), lambda i: (0, 0)),
            pl.BlockSpec((1, _D), lambda i: (0, 0)),
        ],
        out_specs=[half, half],
        out_shape=[jax.ShapeDtypeStruct((_N, _DH), jnp.float32),
                   jax.ShapeDtypeStruct((_N, _DH), jnp.float32)],
    )(jnp.reshape(eps, (1, 1)), hl, hr, al, ar, w1,
      jnp.reshape(b1, (1, _D)), w2, jnp.reshape(b2, (1, _D)))


_CP = 128


def _head_body(rl_ref, rr_ref, wut_ref, wub_ref, wvt_ref, wvb_ref,
               b_ref, lab_ref, logits_ref, loss_ref):
    lp = (jnp.dot(rl_ref[0:_B, :], wut_ref[...],
                  preferred_element_type=jnp.float32)
          + jnp.dot(rr_ref[0:_B, :], wub_ref[...],
                    preferred_element_type=jnp.float32)
          + jnp.dot(rl_ref[_B:2 * _B, :], wvt_ref[...],
                    preferred_element_type=jnp.float32)
          + jnp.dot(rr_ref[_B:2 * _B, :], wvb_ref[...],
                    preferred_element_type=jnp.float32)
          + b_ref[...])
    logits_ref[...] = lp
    lab = lab_ref[...]
    bce = jnp.maximum(lp, 0.0) - lp * lab + jnp.log1p(jnp.exp(-jnp.abs(lp)))
    col = lax.broadcasted_iota(jnp.int32, (_B, _CP), 1)
    bce = jnp.where(col < _C, bce, 0.0)
    loss_ref[...] = jnp.reshape(jnp.sum(bce) / (_B * _C), (1, 1))


def _head(rl, rr, wut, wub, wvt, wvb, b_pad, lab_pad):
    return pl.pallas_call(
        _head_body,
        out_shape=[
            jax.ShapeDtypeStruct((_B, _CP), jnp.float32),
            jax.ShapeDtypeStruct((1, 1), jnp.float32),
        ],
    )(rl, rr, wut, wub, wvt, wvb, b_pad, lab_pad)


def kernel(x, edge_index, edge_attr, ppi, labels,
           eps0, W1_0, b1_0, W2_0, b2_0,
           eps1, W1_1, b1_1, W2_1, b2_1,
           W_out, b_out):
    src = edge_index[0]
    dst = edge_index[1]
    xl = x[:, :_DH]
    xr = x[:, _DH:]

    al0, ar0 = _agg(xl, xr, src, dst, edge_attr)
    hl1, hr1 = _mlp(xl, xr, al0, ar0, eps0, W1_0, b1_0, W2_0, b2_0,
                    relu_out=True)
    al1, ar1 = _agg(hl1, hr1, src, dst, edge_attr)
    hl2, hr2 = _mlp(hl1, hr1, al1, ar1, eps1, W1_1, b1_1, W2_1, b2_1,
                    relu_out=False)

    idx = jnp.concatenate([ppi[:, 0], ppi[:, 1]])
    rl, rr = _pair_gather(hl2, hr2, idx)

    w_pad = jnp.zeros((2 * _D, _CP), jnp.float32).at[:, :_C].set(W_out)
    b_pad = jnp.zeros((1, _CP), jnp.float32).at[0, :_C].set(b_out)
    lab_pad = jnp.zeros((_B, _CP), jnp.float32).at[:, :_C].set(labels)

    logits_pad, loss = _head(rl, rr,
                             w_pad[0:_DH], w_pad[_DH:_D],
                             w_pad[_D:_D + _DH], w_pad[_D + _DH:],
                             b_pad, lab_pad)
    return (logits_pad[:, :_C], loss[0, 0])


# speedup vs baseline: 5.0762x; 2.8695x over previous
import functools

import jax
import jax.numpy as jnp
from jax import lax
from jax.experimental import pallas as pl
from jax.experimental.pallas import tpu as pltpu
from jax.experimental.pallas import tpu_sc as plsc

_N = 10000
_E = 160000
_D = 256
_B = 4096
_C = 7

_NC = 2
_NS = 16
_L = 16

_DH = _D // 2
_SPR = 10016
_TRASH = _N
_G = 64
_EPW = _E // _NS
_FULL = _EPW // _G

_mesh = plsc.VectorSubcoreMesh(
    core_axis_name="c", subcore_axis_name="s",
    num_cores=_NC, num_subcores=_NS)


@functools.partial(
    pl.kernel,
    out_type=[jax.ShapeDtypeStruct((_N, _DH), jnp.float32),
              jax.ShapeDtypeStruct((_N, _DH), jnp.float32)],
    mesh=_mesh,
    scratch_types=[
        pltpu.VMEM((3, _G), jnp.int32),
        pltpu.VMEM((3, _G), jnp.int32),
        pltpu.VMEM((3, _G), jnp.int32),
        pltpu.VMEM((2, _G, _DH), jnp.float32),
        pltpu.VMEM((3, _G, _DH), jnp.float32),
        pltpu.VMEM_SHARED((_SPR, _DH), jnp.float32),
        pltpu.SemaphoreType.DMA((3,)),
        pltpu.SemaphoreType.DMA((3,)),
        pltpu.SemaphoreType.DMA((2,)),
        pltpu.SemaphoreType.DMA((3,)),
        pltpu.SemaphoreType.DMA((3,)),
    ],
)
def _agg(hl_hbm, hr_hbm, src_hbm, dst_hbm, ea_hbm, outl_hbm, outr_hbm,
         srcv, dstv, dscat, hbuf, mbuf, accs,
         ssrc, sdst, gsem, esem, ssem):
    c = lax.axis_index("c")
    s = lax.axis_index("s")
    wbase = s * _EPW

    def _zrow(r, carry):
        for j in range(_DH // _L):
            mbuf[0, r, pl.ds(j * _L, _L)] = jnp.zeros((_L,), jnp.float32)
        return carry
    lax.fori_loop(0, _G, _zrow, 0)
    zoff = s * (_SPR // _NS)
    for k in range(9):
        pltpu.sync_copy(mbuf.at[0], accs.at[pl.ds(zoff + k * _G, _G)])
    pltpu.sync_copy(mbuf.at[0, pl.ds(0, 50)],
                    accs.at[pl.ds(zoff + 9 * _G, 50)])
    plsc.subcore_barrier()

    def _issue_idx(t, i):
        base = wbase + t * _G
        pltpu.async_copy(src_hbm.at[pl.ds(base, _G)], srcv.at[i], ssrc.at[i])
        pltpu.async_copy(dst_hbm.at[pl.ds(base, _G)], dstv.at[i], sdst.at[i])

    def _wait_idx(i):
        pltpu.make_async_copy(src_hbm.at[pl.ds(0, _G)], srcv.at[i],
                              ssrc.at[i]).wait()
        pltpu.make_async_copy(dst_hbm.at[pl.ds(0, _G)], dstv.at[i],
                              sdst.at[i]).wait()

    def _issue_load(t, i, l):
        base = wbase + t * _G

        @pl.when(c == 0)
        def _():
            pltpu.async_copy(hl_hbm.at[srcv.at[i]], hbuf.at[l], gsem.at[l])
            pltpu.async_copy(ea_hbm.at[pl.ds(base, _G), pl.ds(0, _DH)],
                             mbuf.at[i], esem.at[i])

        @pl.when(c == 1)
        def _():
            pltpu.async_copy(hr_hbm.at[srcv.at[i]], hbuf.at[l], gsem.at[l])
            pltpu.async_copy(ea_hbm.at[pl.ds(base, _G), pl.ds(_DH, _DH)],
                             mbuf.at[i], esem.at[i])

    def _wait_load(i, l):
        pltpu.make_async_copy(hl_hbm.at[srcv.at[i]], hbuf.at[l],
                              gsem.at[l]).wait()
        pltpu.make_async_copy(ea_hbm.at[pl.ds(0, _G), pl.ds(0, _DH)],
                              mbuf.at[i], esem.at[i]).wait()

    def _wait_scat(i):
        pltpu.make_async_copy(mbuf.at[i], accs.at[dscat.at[i]],
                              ssem.at[i]).wait()

    def _compute(i, l):
        @plsc.parallel_loop(0, _G, 1, unroll=4)
        def _mrow(r):
            for j in range(_DH // _L):
                sl = pl.ds(j * _L, _L)
                mbuf[i, r, sl] = jnp.maximum(hbuf[l, r, sl] + mbuf[i, r, sl],
                                             0.0)

    pltpu.sync_copy(src_hbm.at[pl.ds(wbase, _G)], srcv.at[0])
    pltpu.sync_copy(dst_hbm.at[pl.ds(wbase, _G)], dstv.at[0])
    _issue_idx(1, 1)
    _issue_load(0, 0, 0)

    def _body(t, carry):
        i0 = lax.rem(t, 3)
        i1 = lax.rem(t + 1, 3)
        i2 = lax.rem(t + 2, 3)
        l0 = lax.rem(t, 2)
        l1 = lax.rem(t + 1, 2)

        @pl.when(t + 2 < _FULL)
        def _():
            _issue_idx(t + 2, i2)

        @pl.when(t + 1 < _FULL)
        def _():
            _wait_idx(i1)

            @pl.when(t >= 2)
            def _():
                _wait_scat(i1)

            _issue_load(t + 1, i1, l1)

        _wait_load(i0, l0)
        _compute(i0, l0)
        for k in range(_G // _L):
            dscat[i0, pl.ds(k * _L, _L)] = dstv[i0, pl.ds(k * _L, _L)]
        pltpu.async_copy(mbuf.at[i0], accs.at[dscat.at[i0]], ssem.at[i0],
                         add=True)
        return carry
    lax.fori_loop(0, _FULL, _body, 0)
    for i in range(3):
        _wait_scat(i)

    tb = _EPW - _G
    pltpu.sync_copy(src_hbm.at[pl.ds(wbase + tb, _G)], srcv.at[0])
    pltpu.sync_copy(dst_hbm.at[pl.ds(wbase + tb, _G)], dstv.at[0])
    tbase = wbase + tb

    @pl.when(c == 0)
    def _():
        pltpu.async_copy(hl_hbm.at[srcv.at[0]], hbuf.at[0], gsem.at[0])
        pltpu.async_copy(ea_hbm.at[pl.ds(tbase, _G), pl.ds(0, _DH)],
                         mbuf.at[0], esem.at[0])

    @pl.when(c == 1)
    def _():
        pltpu.async_copy(hr_hbm.at[srcv.at[0]], hbuf.at[0], gsem.at[0])
        pltpu.async_copy(ea_hbm.at[pl.ds(tbase, _G), pl.ds(_DH, _DH)],
                         mbuf.at[0], esem.at[0])

    _wait_load(0, 0)
    _compute(0, 0)
    for k in range(3):
        dscat[0, pl.ds(k * _L, _L)] = jnp.full((_L,), _TRASH, jnp.int32)
    dscat[0, pl.ds(48, _L)] = dstv[0, pl.ds(48, _L)]
    pltpu.sync_copy(mbuf.at[0], accs.at[dscat.at[0]], add=True)
    plsc.subcore_barrier()

    def _copy_out(out_hbm):
        @pl.when(s < 15)
        def _():
            off = s * 624
            pltpu.sync_copy(accs.at[pl.ds(off, 624)],
                            out_hbm.at[pl.ds(off, 624)])

        @pl.when(s == 15)
        def _():
            pltpu.sync_copy(accs.at[pl.ds(15 * 624, 640)],
                            out_hbm.at[pl.ds(15 * 624, 640)])

    @pl.when(c == 0)
    def _():
        _copy_out(outl_hbm)

    @pl.when(c == 1)
    def _():
        _copy_out(outr_hbm)


_RPW = (2 * _B) // (_NC * _NS)


@functools.partial(
    pl.kernel,
    out_type=[jax.ShapeDtypeStruct((2 * _B, _DH), jnp.float32),
              jax.ShapeDtypeStruct((2 * _B, _DH), jnp.float32)],
    mesh=_mesh,
    scratch_types=[
        pltpu.VMEM((_RPW,), jnp.int32),
        pltpu.VMEM((_RPW, _DH), jnp.float32),
        pltpu.VMEM((_RPW, _DH), jnp.float32),
        pltpu.SemaphoreType.DMA,
        pltpu.SemaphoreType.DMA,
    ],
)
def _pair_gather(hl_hbm, hr_hbm, idx_hbm, outl_hbm, outr_hbm,
                 idxv, rowsl, rowsr, seml, semr):
    c = lax.axis_index("c")
    s = lax.axis_index("s")
    base = (s * _NC + c) * _RPW
    pltpu.sync_copy(idx_hbm.at[pl.ds(base, _RPW)], idxv)
    cl = pltpu.async_copy(hl_hbm.at[idxv], rowsl, seml)
    cr = pltpu.async_copy(hr_hbm.at[idxv], rowsr, semr)
    cl.wait()
    cr.wait()
    pltpu.sync_copy(rowsl, outl_hbm.at[pl.ds(base, _RPW)])
    pltpu.sync_copy(rowsr, outr_hbm.at[pl.ds(base, _RPW)])


_R = 400


def _mlp_body(relu_out, eps_ref, hl_ref, hr_ref, al_ref, ar_ref,
              w1_ref, b1_ref, w2_ref, b2_ref, outl_ref, outr_ref):
    h = jnp.concatenate([hl_ref[...], hr_ref[...]], axis=1)
    agg = jnp.concatenate([al_ref[...], ar_ref[...]], axis=1)
    z = (1.0 + eps_ref[0, 0]) * h + agg
    z = jnp.maximum(
        jnp.dot(z, w1_ref[...], preferred_element_type=jnp.float32)
        + b1_ref[...], 0.0)
    z = jnp.dot(z, w2_ref[...], preferred_element_type=jnp.float32) + b2_ref[...]
    if relu_out:
        z = jnp.maximum(z, 0.0)
    outl_ref[...] = z[:, :_DH]
    outr_ref[...] = z[:, _DH:]


def _mlp(hl, hr, al, ar, eps, w1, b1, w2, b2, relu_out):
    half = pl.BlockSpec((_R, _DH), lambda i: (i, 0))
    return pl.pallas_call(
        functools.partial(_mlp_body, relu_out),
        grid=(_N // _R,),
        in_specs=[
            pl.BlockSpec(memory_space=pltpu.SMEM),
            half, half, half, half,
            pl.BlockSpec((_D, _D), lambda i: (0, 0)),
            pl.BlockSpec((1, _D), lambda i: (0, 0)),
            pl.BlockSpec((_D, _D), lambda i: (0, 0)),
            pl.BlockSpec((1, _D), lambda i: (0, 0)),
        ],
        out_specs=[half, half],
        out_shape=[jax.ShapeDtypeStruct((_N, _DH), jnp.float32),
                   jax.ShapeDtypeStruct((_N, _DH), jnp.float32)],
    )(jnp.reshape(eps, (1, 1)), hl, hr, al, ar, w1,
      jnp.reshape(b1, (1, _D)), w2, jnp.reshape(b2, (1, _D)))


_CP = 128


def _head_body(rl_ref, rr_ref, wut_ref, wub_ref, wvt_ref, wvb_ref,
               b_ref, lab_ref, logits_ref, loss_ref):
    lp = (jnp.dot(rl_ref[0:_B, :], wut_ref[...],
                  preferred_element_type=jnp.float32)
          + jnp.dot(rr_ref[0:_B, :], wub_ref[...],
                    preferred_element_type=jnp.float32)
          + jnp.dot(rl_ref[_B:2 * _B, :], wvt_ref[...],
                    preferred_element_type=jnp.float32)
          + jnp.dot(rr_ref[_B:2 * _B, :], wvb_ref[...],
                    preferred_element_type=jnp.float32)
          + b_ref[...])
    logits_ref[...] = lp
    lab = lab_ref[...]
    bce = jnp.maximum(lp, 0.0) - lp * lab + jnp.log1p(jnp.exp(-jnp.abs(lp)))
    col = lax.broadcasted_iota(jnp.int32, (_B, _CP), 1)
    bce = jnp.where(col < _C, bce, 0.0)
    loss_ref[...] = jnp.reshape(jnp.sum(bce) / (_B * _C), (1, 1))


def _head(rl, rr, wut, wub, wvt, wvb, b_pad, lab_pad):
    return pl.pallas_call(
        _head_body,
        out_shape=[
            jax.ShapeDtypeStruct((_B, _CP), jnp.float32),
            jax.ShapeDtypeStruct((1, 1), jnp.float32),
        ],
    )(rl, rr, wut, wub, wvt, wvb, b_pad, lab_pad)


def kernel(x, edge_index, edge_attr, ppi, labels,
           eps0, W1_0, b1_0, W2_0, b2_0,
           eps1, W1_1, b1_1, W2_1, b2_1,
           W_out, b_out):
    src = edge_index[0]
    dst = edge_index[1]
    xl = x[:, :_DH]
    xr = x[:, _DH:]

    al0, ar0 = _agg(xl, xr, src, dst, edge_attr)
    hl1, hr1 = _mlp(xl, xr, al0, ar0, eps0, W1_0, b1_0, W2_0, b2_0,
                    relu_out=True)
    al1, ar1 = _agg(hl1, hr1, src, dst, edge_attr)
    hl2, hr2 = _mlp(hl1, hr1, al1, ar1, eps1, W1_1, b1_1, W2_1, b2_1,
                    relu_out=False)

    idx = jnp.concatenate([ppi[:, 0], ppi[:, 1]])
    rl, rr = _pair_gather(hl2, hr2, idx)

    w_pad = jnp.zeros((2 * _D, _CP), jnp.float32).at[:, :_C].set(W_out)
    b_pad = jnp.zeros((1, _CP), jnp.float32).at[0, :_C].set(b_out)
    lab_pad = jnp.zeros((_B, _CP), jnp.float32).at[:, :_C].set(labels)

    logits_pad, loss = _head(rl, rr,
                             w_pad[0:_DH], w_pad[_DH:_D],
                             w_pad[_D:_D + _DH], w_pad[_D + _DH:],
                             b_pad, lab_pad)
    return (logits_pad[:, :_C], loss[0, 0])
